# baseline (speedup 1.0000x reference)
import jax
import jax.numpy as jnp
from jax import lax
from jax.experimental import pallas as pl
from jax.experimental.pallas import tpu as pltpu
from jax.experimental.pallas import tpu_sc as plsc

_H = 1000
_C = 128
_NC = 2
_NS = 16
_NW = _NC * _NS
_CHUNK = 32


def _sc_body(table_hbm, *rest):
    outs = rest[:_H]
    rows_v = rest[_H]
    sem = rest[_H + 1]
    wid = lax.axis_index("s") * _NC + lax.axis_index("c")
    for w in range(_NW):
        base = w * _CHUNK
        n = min(_CHUNK, _H - base)

        @pl.when(wid == w)
        def _(base=base, n=n):
            pltpu.sync_copy(table_hbm.at[pl.ds(base, n)], rows_v.at[pl.ds(0, n)])
            copies = [
                pltpu.make_async_copy(rows_v.at[pl.ds(j, 1)], outs[base + j], sem)
                for j in range(n)
            ]
            for c in copies:
                c.start()
            for c in copies:
                c.wait()


def kernel(table):
    outs = pl.kernel(
        _sc_body,
        out_type=[jax.ShapeDtypeStruct((1, _C), jnp.float32)] * _H,
        mesh=plsc.VectorSubcoreMesh(core_axis_name="c", subcore_axis_name="s"),
        scratch_types=[
            pltpu.VMEM((_CHUNK, _C), jnp.float32),
            pltpu.SemaphoreType.DMA,
        ],
    )(table)
    return {f"embeds_bb_{i}": {"codes": outs[i]} for i in range(_H)}


# speedup vs baseline: 5.7005x; 5.7005x over previous
import jax
import jax.numpy as jnp
from jax.experimental import pallas as pl
from jax.experimental.pallas import tpu as pltpu

_H = 1000
_C = 128
_NSEM = 8


def _copy_rows_body(table_ref, *rest):
    outs = rest[:_H]
    vmem = rest[_H]
    sem_in = rest[_H + 1]
    sems = rest[_H + 2:_H + 2 + _NSEM]
    pltpu.make_async_copy(table_ref, vmem, sem_in).start()
    pltpu.make_async_copy(table_ref, vmem, sem_in).wait()
    copies = [
        pltpu.make_async_copy(vmem.at[pl.ds(i, 1)], outs[i], sems[i % _NSEM])
        for i in range(_H)
    ]
    for c in copies:
        c.start()
    for c in copies:
        c.wait()


def kernel(table):
    outs = pl.pallas_call(
        _copy_rows_body,
        in_specs=[pl.BlockSpec(memory_space=pl.ANY)],
        out_specs=[pl.BlockSpec(memory_space=pl.ANY)] * _H,
        out_shape=[jax.ShapeDtypeStruct((1, _C), jnp.float32)] * _H,
        scratch_shapes=[pltpu.VMEM((_H, _C), jnp.float32)]
        + [pltpu.SemaphoreType.DMA] * (1 + _NSEM),
    )(table)
    return {f"embeds_bb_{i}": {"codes": outs[i]} for i in range(_H)}
